# baseline (speedup 1.0000x reference)
import functools

import jax
import jax.numpy as jnp
from jax import lax
from jax.experimental import pallas as pl
from jax.experimental.pallas import tpu as pltpu
from jax.experimental.pallas import tpu_sc as plsc

_B = 1024
_NB = 14
_LANES = 16
_NUM_CORES = 2
_NUM_SUBCORES = 16
_NUM_TECS = _NUM_CORES * _NUM_SUBCORES
_GROUP = 8
_PW = 512
_PACK_BLK = 64


def _pack_body(m_ref, o_ref):
  m = m_ref[...].astype(jnp.int32).reshape(_PACK_BLK, 32, _PW)
  shifts = jnp.arange(32, dtype=jnp.int32)[None, :, None]
  o_ref[...] = jnp.sum(m << shifts, axis=1)


def _pack_bits(mem_bool):
  n = mem_bool.shape[0]
  return pl.pallas_call(
      _pack_body,
      grid=(n // _PACK_BLK,),
      in_specs=[pl.BlockSpec((_PACK_BLK, 32 * _PW), lambda i: (i, 0))],
      out_specs=pl.BlockSpec((_PACK_BLK, _PW), lambda i: (i, 0)),
      out_shape=jax.ShapeDtypeStruct((n, _PW), jnp.int32),
  )(mem_bool)


def _ram_layer_kernel(n_neurons, n_cols, xT, connp, memp, out,
                      conn_v, planes_a, planes_b, mem_a, mem_b, out_v,
                      psem_a, psem_b, msem_a, msem_b):
  n_per_tec = n_neurons // _NUM_TECS
  n_groups = n_per_tec // _GROUP
  n_pairs = n_groups // 2
  cid = lax.axis_index("c")
  sid = lax.axis_index("s")
  wid = sid * _NUM_CORES + cid
  n0 = wid * n_per_tec

  pltpu.sync_copy(connp.at[pl.ds(n0 * 16, n_per_tec * 16)], conn_v)

  def fire(g, planes, mem, psem, msem):
    idx = conn_v.at[pl.ds(g * _GROUP * 16, _GROUP * 16)]
    pltpu.async_copy(xT.at[idx], planes, psem)
    pltpu.async_copy(memp.at[wid * n_groups + g], mem, msem)

  def drain(planes, mem, psem, msem):
    pltpu.make_async_copy(xT.at[pl.ds(0, _GROUP * 16)], planes, psem).wait()
    pltpu.make_async_copy(memp.at[0], mem, msem).wait()

  def compute(g, planes, mem):
    def vec_body(v, c):
      for j in range(_GROUP):
        ps = [planes[j * 16 + k, pl.ds(v * _LANES, _LANES)]
              for k in range(_NB)]
        quads = []
        for k0 in range(0, 12, 4):
          q = (ps[k0] + (ps[k0 + 1] << 1)
               + (ps[k0 + 2] << 2) + (ps[k0 + 3] << 3))
          quads.append((k0, q, 0xF))
        quads.append((12, ps[12] + (ps[13] << 1), 0x3))
        word = None
        for t in range(4):
          a = None
          for k0, q, m in quads:
            s = 8 * t - k0
            if s >= 0:
              term = (q >> s) & (m << k0)
            else:
              term = (q << (-s)) & (m << k0)
            a = term if a is None else a + term
          w = plsc.load_gather(mem, [(a & (_PW - 1)) + j * _PW])
          bit = (w >> (a >> 9)) & 1
          bit = bit << (8 * t) if t else bit
          word = bit if word is None else word | bit
        out_v[pl.ds(j * (_B // 4) + v * _LANES, _LANES)] = word
      return c

    lax.fori_loop(0, _B // 64, vec_body, 0, unroll=2)
    pltpu.sync_copy(out_v, out.at[wid * n_groups + g])

  fire(0, planes_a, mem_a, psem_a, msem_a)

  def pair_body(p, c):
    g0 = 2 * p
    fire(g0 + 1, planes_b, mem_b, psem_b, msem_b)
    drain(planes_a, mem_a, psem_a, msem_a)
    compute(g0, planes_a, mem_a)

    @pl.when(p + 1 < n_pairs)
    def _prefetch():
      fire(g0 + 2, planes_a, mem_a, psem_a, msem_a)

    drain(planes_b, mem_b, psem_b, msem_b)
    compute(g0 + 1, planes_b, mem_b)
    return c

  lax.fori_loop(0, n_pairs, pair_body, 0)


def _ram_layer(xT, connp, memp, n_neurons):
  n_cols = xT.shape[0]
  n_per_tec = n_neurons // _NUM_TECS
  mesh = plsc.VectorSubcoreMesh(
      core_axis_name="c", subcore_axis_name="s",
      num_cores=_NUM_CORES, num_subcores=_NUM_SUBCORES)
  body = functools.partial(_ram_layer_kernel, n_neurons, n_cols)
  f = pl.kernel(
      body,
      out_type=jax.ShapeDtypeStruct(
          (n_neurons // _GROUP, _GROUP * (_B // 4)), jnp.int32),
      mesh=mesh,
      compiler_params=pltpu.CompilerParams(needs_layout_passes=False),
      scratch_types=[
          pltpu.VMEM((n_per_tec * 16,), jnp.int32),
          pltpu.VMEM((_GROUP * 16, _B // 4), jnp.int32),
          pltpu.VMEM((_GROUP * 16, _B // 4), jnp.int32),
          pltpu.VMEM((_GROUP * _PW,), jnp.int32),
          pltpu.VMEM((_GROUP * _PW,), jnp.int32),
          pltpu.VMEM((_GROUP * (_B // 4),), jnp.int32),
          pltpu.SemaphoreType.DMA,
          pltpu.SemaphoreType.DMA,
          pltpu.SemaphoreType.DMA,
          pltpu.SemaphoreType.DMA,
      ],
      name=f"ram_layer_n{n_neurons}",
  )
  memp_g = memp.reshape(n_neurons // _GROUP, _GROUP * _PW)
  return f(xT, connp.reshape(-1), memp_g).reshape(n_neurons, _B // 4)


def kernel(input_bits, prev_state_bits, in_conn, in_mem, st_conn, st_mem):
  def _to_planes(bits_T):
    t = bits_T.shape[0]
    return lax.bitcast_convert_type(
        bits_T.astype(jnp.int8).reshape(t, _B // 4, 4), jnp.int32)

  x = jnp.concatenate([input_bits, prev_state_bits], axis=1)
  xT = _to_planes(x.T)
  prevT = _to_planes(prev_state_bits.T)

  in_connp = jnp.pad(in_conn.astype(jnp.int32), ((0, 0), (0, 2)))
  st_connp = jnp.pad(st_conn.astype(jnp.int32), ((0, 0), (0, 2)))
  in_memp = _pack_bits(in_mem)
  st_memp = _pack_bits(st_mem)

  outT1 = _ram_layer(xT, in_connp, in_memp, in_conn.shape[0])
  yT = jnp.concatenate([outT1, prevT], axis=0)
  outT2 = _ram_layer(yT, st_connp, st_memp, st_conn.shape[0])

  def _from_planes(planes, n):
    b = lax.bitcast_convert_type(planes, jnp.int8).reshape(n, _B)
    return b.T.astype(bool)

  input_out = _from_planes(outT1, in_conn.shape[0])
  next_state = _from_planes(outT2, st_conn.shape[0])
  return (input_out, next_state)


# speedup vs baseline: 1.6692x; 1.6692x over previous
import functools

import jax
import jax.numpy as jnp
from jax import lax
from jax.experimental import pallas as pl
from jax.experimental.pallas import tpu as pltpu
from jax.experimental.pallas import tpu_sc as plsc

_B = 1024
_NB = 14
_LANES = 16
_NUM_CORES = 2
_NUM_SUBCORES = 16
_NUM_TECS = _NUM_CORES * _NUM_SUBCORES
_GROUP = 8
_PW = 512
_PACK_BLK = 64


def _pack_body(w, m_ref, o_ref):
  m = m_ref[...].astype(jnp.int32).reshape(_PACK_BLK, 32, w)
  shifts = jnp.arange(32, dtype=jnp.int32)[None, :, None]
  o_ref[...] = jnp.sum(m << shifts, axis=1)


def _pack_bits(bits, w):
  n = bits.shape[0]
  return pl.pallas_call(
      functools.partial(_pack_body, w),
      grid=(n // _PACK_BLK,),
      in_specs=[pl.BlockSpec((_PACK_BLK, 32 * w), lambda i: (i, 0))],
      out_specs=pl.BlockSpec((_PACK_BLK, w), lambda i: (i, 0)),
      out_shape=jax.ShapeDtypeStruct((n, w), jnp.int32),
  )(bits)


def _ram_layer_kernel(n_neurons, n_cols, xp, connp, memp, out,
                      conn_v, mem_a, mem_b, out_v, msem_a, msem_b):
  n_per_tec = n_neurons // _NUM_TECS
  n_groups = n_per_tec // _GROUP
  cid = lax.axis_index("c")
  sid = lax.axis_index("s")
  wid = sid * _NUM_CORES + cid
  n0 = wid * n_per_tec

  def scoped(xs_v):
    _ram_layer_inner(n_neurons, n_cols, xp, connp, memp, out, xs_v,
                     conn_v, mem_a, mem_b, out_v, msem_a, msem_b)

  pl.run_scoped(scoped, pltpu.VMEM((n_cols * 32,), jnp.int32))


def _ram_layer_inner(n_neurons, n_cols, xp, connp, memp, out, xs_v,
                     conn_v, mem_a, mem_b, out_v, msem_a, msem_b):
  n_per_tec = n_neurons // _NUM_TECS
  n_groups = n_per_tec // _GROUP
  cid = lax.axis_index("c")
  sid = lax.axis_index("s")
  wid = sid * _NUM_CORES + cid
  n0 = wid * n_per_tec

  pltpu.sync_copy(xp, xs_v)
  pltpu.sync_copy(connp.at[pl.ds(n0 * 16, n_per_tec * 16)], conn_v)
  del n_cols

  def fire(g, mem, msem):
    pltpu.async_copy(memp.at[wid * n_groups + g], mem, msem)

  def drain(mem, msem):
    pltpu.make_async_copy(memp.at[0], mem, msem).wait()

  def compute(g, mem):
    for j in range(_GROUP):
      cbase = (g * _GROUP + j) * 16
      cvec = conn_v[pl.ds(cbase, _LANES)]
      for half in range(2):
        ps = [xs_v[pl.ds(cvec[k] * 32 + half * _LANES, _LANES)]
              for k in range(_NB)]

        def bit_body(jb, ow, ps=ps, j=j):
          a = None
          for k in range(_NB):
            term = ((ps[k] >> jb) & 1) << k
            a = term if a is None else a | term
          w = plsc.load_gather(mem, [(a & (_PW - 1)) + j * _PW])
          bit = (w >> (a >> 9)) & 1
          return ow | (bit << jb)

        ow = lax.fori_loop(0, 32, bit_body, jnp.zeros((_LANES,), jnp.int32),
                           unroll=4)
        out_v[pl.ds(j * 32 + half * _LANES, _LANES)] = ow
    pltpu.sync_copy(out_v, out.at[wid * n_groups + g])

  fire(0, mem_a, msem_a)

  def pair_body(p, c):
    g0 = 2 * p
    fire(g0 + 1, mem_b, msem_b)
    drain(mem_a, msem_a)
    compute(g0, mem_a)

    @pl.when(p + 1 < n_groups // 2)
    def _prefetch():
      fire(g0 + 2, mem_a, msem_a)

    drain(mem_b, msem_b)
    compute(g0 + 1, mem_b)
    return c

  lax.fori_loop(0, n_groups // 2, pair_body, 0)


def _ram_layer(xp, connp, memp, n_neurons):
  n_cols = xp.shape[0]
  n_per_tec = n_neurons // _NUM_TECS
  mesh = plsc.VectorSubcoreMesh(
      core_axis_name="c", subcore_axis_name="s",
      num_cores=_NUM_CORES, num_subcores=_NUM_SUBCORES)
  body = functools.partial(_ram_layer_kernel, n_neurons, n_cols)
  f = pl.kernel(
      body,
      out_type=jax.ShapeDtypeStruct(
          (n_neurons // _GROUP, _GROUP * 32), jnp.int32),
      mesh=mesh,
      compiler_params=pltpu.CompilerParams(needs_layout_passes=False),
      scratch_types=[
          pltpu.VMEM((n_per_tec * 16,), jnp.int32),
          pltpu.VMEM((_GROUP * _PW,), jnp.int32),
          pltpu.VMEM((_GROUP * _PW,), jnp.int32),
          pltpu.VMEM((_GROUP * 32,), jnp.int32),
          pltpu.SemaphoreType.DMA,
          pltpu.SemaphoreType.DMA,
      ],
      name=f"ram_layer_n{n_neurons}",
  )
  memp_g = memp.reshape(n_neurons // _GROUP, _GROUP * _PW)
  return f(xp.reshape(-1), connp.reshape(-1), memp_g).reshape(n_neurons, 32)


def kernel(input_bits, prev_state_bits, in_conn, in_mem, st_conn, st_mem):
  x = jnp.concatenate([input_bits, prev_state_bits], axis=1)
  xp = _pack_bits(x.T, 32)
  prevp = _pack_bits(prev_state_bits.T, 32)

  in_connp = jnp.pad(in_conn.astype(jnp.int32), ((0, 0), (0, 2)))
  st_connp = jnp.pad(st_conn.astype(jnp.int32), ((0, 0), (0, 2)))
  in_memp = _pack_bits(in_mem, _PW)
  st_memp = _pack_bits(st_mem, _PW)

  out1 = _ram_layer(xp, in_connp, in_memp, in_conn.shape[0])
  yp = jnp.concatenate([out1, prevp], axis=0)
  out2 = _ram_layer(yp, st_connp, st_memp, st_conn.shape[0])

  def _unpack(words, n):
    bits = (words[:, None, :] >> jnp.arange(32, dtype=jnp.int32)[None, :, None]) & 1
    return bits.reshape(n, _B).T.astype(bool)

  input_out = _unpack(out1, in_conn.shape[0])
  next_state = _unpack(out2, st_conn.shape[0])
  return (input_out, next_state)
